# baseline (speedup 1.0000x reference)
import math
import jax
import jax.numpy as jnp
from jax import lax
from jax.experimental import pallas as pl
from jax.experimental.pallas import tpu as pltpu

B, N, D, H, L = 256, 101, 128, 8, 3
FF = 2048
NP = 128
DH = D // H
MAX_STEPS = 2 * (N - 1) + 1
SQRT_D = math.sqrt(D)
HIGH = lax.Precision.HIGHEST
BB = 16
PB = 32


def _ln(x, g, b):
    mu = jnp.mean(x, -1, keepdims=True)
    xc = x - mu
    var = jnp.mean(xc * xc, -1, keepdims=True)
    return xc / jnp.sqrt(var + 1e-5) * g + b


def _enc_kernel(x_ref, wemb_ref, bemb_ref, wqkv_ref, bqkv_ref, wo_ref, bo_ref,
                w1_ref, b1_ref, w2_ref, b2_ref, ln1g_ref, ln1b_ref,
                ln2g_ref, ln2b_ref, h_out_ref):
    M = BB * NP
    x = x_ref[...].reshape(M, 8)
    h = jnp.dot(x, wemb_ref[...]) + bemb_ref[...]
    jmask = lax.broadcasted_iota(jnp.int32, (1, 1, NP), 2) >= N
    for l in range(L):
        qkv = jnp.dot(h, wqkv_ref[l]) + bqkv_ref[l:l + 1, :]
        q3 = qkv[:, 0:D].reshape(BB, NP, D)
        k3 = qkv[:, D:2 * D].reshape(BB, NP, D)
        v3 = qkv[:, 2 * D:3 * D].reshape(BB, NP, D)
        o_parts = []
        for hd in range(H):
            sl = slice(hd * DH, (hd + 1) * DH)
            qh = q3[:, :, sl]
            kh = k3[:, :, sl]
            vh = v3[:, :, sl]
            s = lax.dot_general(qh, kh, (((2,), (2,)), ((0,), (0,))),
                               ) / 4.0
            s = jnp.where(jmask, -1e30, s)
            m = jnp.max(s, axis=-1, keepdims=True)
            e = jnp.exp(s - m)
            att = e / jnp.sum(e, axis=-1, keepdims=True)
            o_parts.append(lax.dot_general(att, vh, (((2,), (1,)), ((0,), (0,))),
                                          ))
        o = jnp.concatenate(o_parts, axis=-1).reshape(M, D)
        o = jnp.dot(o, wo_ref[l]) + bo_ref[l:l + 1, :]
        h = _ln(h + o, ln1g_ref[l:l + 1, :], ln1b_ref[l:l + 1, :])
        ff1 = jnp.maximum(jnp.dot(h, w1_ref[l]) + b1_ref[l:l + 1, :], 0.0)
        ff = jnp.dot(ff1, w2_ref[l]) + b2_ref[l:l + 1, :]
        h = _ln(h + ff, ln2g_ref[l:l + 1, :], ln2b_ref[l:l + 1, :])
    h3 = h.reshape(BB, NP, D)
    rmask = lax.broadcasted_iota(jnp.int32, (1, NP, 1), 1) < N
    h_out_ref[...] = jnp.where(rmask, h3, 0.0)


def _pre_kernel(h_ref, wc1t_ref, wc2t_ref, wc3t_ref, wqt_ref, wk_ref,
                bcap_ref, bctx_ref, wcap_ref,
                t2_ref, hu_ref, hv_ref,
                m1t_s, m2t_s, vrow_s):
    i = pl.program_id(0)

    @pl.when(i == 0)
    def _init():
        p = jnp.dot(wqt_ref[...], wk_ref[...])
        m1t_s[...] = jnp.dot(wc1t_ref[...], p)
        m2t_s[...] = jnp.dot(wc2t_ref[...], p)
        y = (jnp.dot(bcap_ref[...], wc3t_ref[...])
             + bctx_ref[...])
        c0 = jnp.dot(y, p)
        tw = jnp.dot(wcap_ref[...], wc3t_ref[...])
        m3w = jnp.dot(tw, p)
        vrow_s[0:1, :] = c0
        vrow_s[1:2, :] = m3w

    h = h_ref[...]
    gctx = jnp.sum(h, axis=1) / float(N)
    z = jnp.dot(gctx, m2t_s[...]) + vrow_s[0:1, :]
    u2 = jnp.dot(h.reshape(PB * NP, D), m1t_s[...],
                ).reshape(PB, NP, D)
    t2_ref[...] = lax.dot_general(u2, h, (((2,), (2,)), ((0,), (0,))),
                                 )
    hu_ref[...] = jnp.sum(h * vrow_s[1:2, :][None], axis=-1)
    hv_ref[...] = jnp.sum(h * z[:, None, :], axis=-1)


def _dec_kernel(t2_ref, hu_ref, hv_ref, dr_ref, g_ref,
                acts_ref, logps_ref,
                onehot_s, visited_s, remaining_s, prevdep_s, done_s, lacc_s):
    t = pl.program_id(0)
    iota = lax.broadcasted_iota(jnp.int32, (B, NP), 1)
    col0 = iota == 0

    @pl.when(t == 0)
    def _init():
        onehot_s[...] = jnp.where(col0, 1.0, 0.0)
        visited_s[...] = (iota >= N).astype(jnp.float32)
        remaining_s[...] = jnp.ones((B, 1), jnp.float32)
        prevdep_s[...] = jnp.ones((B, 1), jnp.float32)
        done_s[...] = jnp.zeros((B, 1), jnp.float32)
        lacc_s[...] = jnp.zeros((B, 1), jnp.float32)

    dr = dr_ref[...]
    g = g_ref[0]
    visited = visited_s[...]
    remaining = remaining_s[...]
    done_b = done_s[...] > 0.5
    prevdep_b = prevdep_s[...] > 0.5
    oh_prev = onehot_s[...]

    tm = jnp.sum(t2_ref[...] * oh_prev[:, :, None], axis=1)
    logits = (tm + remaining * hu_ref[...] + hv_ref[...]) / SQRT_D

    padc = iota >= N
    nv_c = (1.0 - visited) * jnp.where(col0, 0.0, 1.0)
    all_served = jnp.max(nv_c, axis=1, keepdims=True) < 0.5
    feas = jnp.where(dr <= remaining, 1.0, 0.0) * nv_c
    has_feas = jnp.max(feas, axis=1, keepdims=True) > 0.5

    atdep_b = prevdep_b & (~done_b)
    notdep_b = (~prevdep_b) & (~done_b)
    infeasible = (visited > 0.5) | (dr > remaining)
    mask_depot = atdep_b & has_feas
    infeasible = infeasible | (mask_depot & col0)
    force = (all_served & notdep_b) | (notdep_b & (~has_feas)) | done_b

    ml = jnp.where(padc, -1e30, jnp.where(infeasible, -1e9, logits))
    mx = jnp.max(ml, axis=-1, keepdims=True)
    sh = ml - mx
    logp = sh - jnp.log(jnp.sum(jnp.exp(sh), axis=-1, keepdims=True))

    vals = g + ml
    vm = jnp.max(vals, axis=-1, keepdims=True)
    iota_f = iota.astype(jnp.float32)
    sel_f = jnp.min(jnp.where(vals == vm, iota_f, float(NP)), axis=-1,
                    keepdims=True)
    sel_f = jnp.where(force, 0.0, sel_f)
    onehot = iota_f == sel_f
    sel_logp = jnp.sum(jnp.where(onehot, logp, 0.0), axis=-1, keepdims=True)
    sel_logp = jnp.where(force, 0.0, sel_logp)

    isdep = sel_f < 0.5
    take = jnp.sum(jnp.where(onehot, dr, 0.0), axis=-1, keepdims=True)
    remaining_s[...] = jnp.where(isdep, 1.0, remaining - take)
    visited_s[...] = jnp.maximum(
        visited, jnp.where(onehot & (~isdep), 1.0, 0.0))
    onehot_s[...] = jnp.where(onehot, 1.0, 0.0)
    prevdep_s[...] = jnp.where(isdep, 1.0, 0.0)
    done_s[...] = jnp.where(done_b | (all_served & isdep), 1.0, 0.0)
    lacc = lacc_s[...] + sel_logp
    lacc_s[...] = lacc

    acts_ref[...] = sel_f.astype(jnp.int32).reshape(1, B, 1)
    logps_ref[...] = lacc


def _whole(shape):
    nd = len(shape)
    return pl.BlockSpec(shape, lambda *_: (0,) * nd)


def kernel(coords, demands_raw, capacity_raw, params):
    cap = capacity_raw.reshape(B, 1)
    demand_ratio = demands_raw / cap

    x3 = jnp.concatenate([coords, demand_ratio[..., None]], axis=-1)
    x3p = jnp.zeros((B, NP, 8), jnp.float32).at[:, :N, :3].set(x3)
    wemb = jnp.pad(params['W_embed'], ((0, 0), (0, 5))).T
    bemb = params['b_embed'].reshape(1, D)
    wqkv_t = jnp.transpose(params['Wqkv'], (0, 2, 1))
    wo_t = jnp.transpose(params['Wo'], (0, 2, 1))
    w1_t = jnp.transpose(params['W1'], (0, 2, 1))
    w2_t = jnp.transpose(params['W2'], (0, 2, 1))

    h_pad = pl.pallas_call(
        _enc_kernel,
        grid=(B // BB,),
        in_specs=[
            pl.BlockSpec((BB, NP, 8), lambda i: (i, 0, 0)),
            _whole((8, D)), _whole((1, D)),
            _whole((L, D, 3 * D)), _whole((L, 3 * D)),
            _whole((L, D, D)), _whole((L, D)),
            _whole((L, D, FF)), _whole((L, FF)),
            _whole((L, FF, D)), _whole((L, D)),
            _whole((L, D)), _whole((L, D)),
            _whole((L, D)), _whole((L, D)),
        ],
        out_specs=pl.BlockSpec((BB, NP, D), lambda i: (i, 0, 0)),
        out_shape=jax.ShapeDtypeStruct((B, NP, D), jnp.float32),
    )(x3p, wemb, bemb, wqkv_t, params['bqkv'], wo_t, params['bo'],
      w1_t, params['b1'], w2_t, params['b2'],
      params['ln1_g'], params['ln1_b'], params['ln2_g'], params['ln2_b'])

    t2, hu, hv = pl.pallas_call(
        _pre_kernel,
        grid=(B // PB,),
        in_specs=[
            pl.BlockSpec((PB, NP, D), lambda i: (i, 0, 0)),
            _whole((D, D)), _whole((D, D)), _whole((D, D)),
            _whole((D, D)), _whole((D, D)),
            _whole((1, D)), _whole((1, D)), _whole((1, D)),
        ],
        out_specs=[
            pl.BlockSpec((PB, NP, NP), lambda i: (i, 0, 0)),
            pl.BlockSpec((PB, NP), lambda i: (i, 0)),
            pl.BlockSpec((PB, NP), lambda i: (i, 0)),
        ],
        out_shape=[
            jax.ShapeDtypeStruct((B, NP, NP), jnp.float32),
            jax.ShapeDtypeStruct((B, NP), jnp.float32),
            jax.ShapeDtypeStruct((B, NP), jnp.float32),
        ],
        scratch_shapes=[
            pltpu.VMEM((D, D), jnp.float32),
            pltpu.VMEM((D, D), jnp.float32),
            pltpu.VMEM((8, D), jnp.float32),
        ],
    )(h_pad, params['Wctx'][:, 0:D].T, params['Wctx'][:, D:2 * D].T,
      params['Wctx'][:, 2 * D:3 * D].T, params['Wq'].T, params['Wk'],
      params['bcap'].reshape(1, D), params['bctx'].reshape(1, D),
      params['Wcap'].T)

    keys = jax.random.split(jax.random.key(42), MAX_STEPS)
    G = jax.vmap(lambda k: jax.random.gumbel(k, (B, N), jnp.float32))(keys)
    G_pad = jnp.zeros((MAX_STEPS, B, NP), jnp.float32).at[:, :, :N].set(G)
    dr_pad = jnp.pad(demand_ratio, ((0, 0), (0, NP - N)), constant_values=2.0)

    acts, logps = pl.pallas_call(
        _dec_kernel,
        grid=(MAX_STEPS,),
        in_specs=[
            _whole((B, NP, NP)),
            _whole((B, NP)), _whole((B, NP)), _whole((B, NP)),
            pl.BlockSpec((1, B, NP), lambda t: (t, 0, 0)),
        ],
        out_specs=[
            pl.BlockSpec((1, B, 1), lambda t: (t, 0, 0)),
            pl.BlockSpec((B, 1), lambda t: (0, 0)),
        ],
        out_shape=[
            jax.ShapeDtypeStruct((MAX_STEPS, B, 1), jnp.int32),
            jax.ShapeDtypeStruct((B, 1), jnp.float32),
        ],
        scratch_shapes=[
            pltpu.VMEM((B, NP), jnp.float32),
            pltpu.VMEM((B, NP), jnp.float32),
            pltpu.VMEM((B, 1), jnp.float32),
            pltpu.VMEM((B, 1), jnp.float32),
            pltpu.VMEM((B, 1), jnp.float32),
            pltpu.VMEM((B, 1), jnp.float32),
        ],
    )(t2, hu, hv, dr_pad, G_pad)

    actions = acts.reshape(MAX_STEPS, B).T
    path = jnp.concatenate([jnp.zeros((B, 1), actions.dtype), actions], axis=1)
    return path, logps.reshape(B)


# speedup vs baseline: 1.0630x; 1.0630x over previous
import math
import jax
import jax.numpy as jnp
from jax import lax
from jax.experimental import pallas as pl
from jax.experimental.pallas import tpu as pltpu

B, N, D, H, L = 256, 101, 128, 8, 3
FF = 2048
NP = 128
DH = D // H
MAX_STEPS = 2 * (N - 1) + 1
SQRT_D = math.sqrt(D)
HIGH = lax.Precision.HIGHEST
BB = 32
PB = 32


def _ln(x, g, b):
    mu = jnp.mean(x, -1, keepdims=True)
    xc = x - mu
    var = jnp.mean(xc * xc, -1, keepdims=True)
    return xc / jnp.sqrt(var + 1e-5) * g + b


def _enc_kernel(x_ref, wemb_ref, bemb_ref, wqkv_ref, bqkv_ref, wo_ref, bo_ref,
                w1_ref, b1_ref, w2_ref, b2_ref, ln1g_ref, ln1b_ref,
                ln2g_ref, ln2b_ref, h_out_ref):
    M = BB * NP
    x = x_ref[...].reshape(M, 8)
    h = jnp.dot(x, wemb_ref[...]) + bemb_ref[...]
    jmask = lax.broadcasted_iota(jnp.int32, (1, 1, NP), 2) >= N
    for l in range(L):
        qkv = jnp.dot(h, wqkv_ref[l]) + bqkv_ref[l:l + 1, :]
        q3 = qkv[:, 0:D].reshape(BB, NP, D)
        k3 = qkv[:, D:2 * D].reshape(BB, NP, D)
        v3 = qkv[:, 2 * D:3 * D].reshape(BB, NP, D)
        o_parts = []
        for hd in range(H):
            sl = slice(hd * DH, (hd + 1) * DH)
            qh = q3[:, :, sl]
            kh = k3[:, :, sl]
            vh = v3[:, :, sl]
            s = lax.dot_general(qh, kh, (((2,), (2,)), ((0,), (0,))),
                               ) / 4.0
            s = jnp.where(jmask, -1e30, s)
            m = jnp.max(s, axis=-1, keepdims=True)
            e = jnp.exp(s - m)
            att = e / jnp.sum(e, axis=-1, keepdims=True)
            o_parts.append(lax.dot_general(att, vh, (((2,), (1,)), ((0,), (0,))),
                                          ))
        o = jnp.concatenate(o_parts, axis=-1).reshape(M, D)
        o = jnp.dot(o, wo_ref[l]) + bo_ref[l:l + 1, :]
        h = _ln(h + o, ln1g_ref[l:l + 1, :], ln1b_ref[l:l + 1, :])
        ff = b2_ref[l:l + 1, :].astype(jnp.float32) + jnp.zeros((M, D), jnp.float32)
        for c in range(2):
            cs = slice(c * (FF // 2), (c + 1) * (FF // 2))
            ff1c = jnp.maximum(jnp.dot(h, w1_ref[l][:, cs])
                               + b1_ref[l:l + 1, cs], 0.0)
            ff = ff + jnp.dot(ff1c, w2_ref[l][cs, :])
        h = _ln(h + ff, ln2g_ref[l:l + 1, :], ln2b_ref[l:l + 1, :])
    h3 = h.reshape(BB, NP, D)
    rmask = lax.broadcasted_iota(jnp.int32, (1, NP, 1), 1) < N
    h_out_ref[...] = jnp.where(rmask, h3, 0.0)


def _pre_kernel(h_ref, wc1t_ref, wc2t_ref, wc3t_ref, wqt_ref, wk_ref,
                bcap_ref, bctx_ref, wcap_ref,
                t2_ref, hu_ref, hv_ref,
                m1t_s, m2t_s, vrow_s):
    i = pl.program_id(0)

    @pl.when(i == 0)
    def _init():
        p = jnp.dot(wqt_ref[...], wk_ref[...])
        m1t_s[...] = jnp.dot(wc1t_ref[...], p)
        m2t_s[...] = jnp.dot(wc2t_ref[...], p)
        y = (jnp.dot(bcap_ref[...], wc3t_ref[...])
             + bctx_ref[...])
        c0 = jnp.dot(y, p)
        tw = jnp.dot(wcap_ref[...], wc3t_ref[...])
        m3w = jnp.dot(tw, p)
        vrow_s[0:1, :] = c0
        vrow_s[1:2, :] = m3w

    h = h_ref[...]
    gctx = jnp.sum(h, axis=1) / float(N)
    z = jnp.dot(gctx, m2t_s[...]) + vrow_s[0:1, :]
    u2 = jnp.dot(h.reshape(PB * NP, D), m1t_s[...],
                ).reshape(PB, NP, D)
    t2_ref[...] = lax.dot_general(u2, h, (((2,), (2,)), ((0,), (0,))),
                                 )
    hu_ref[...] = jnp.sum(h * vrow_s[1:2, :][None], axis=-1)
    hv_ref[...] = jnp.sum(h * z[:, None, :], axis=-1)


def _dec_kernel(t2_ref, hu_ref, hv_ref, dr_ref, g_ref,
                acts_ref, logps_ref):
    iota = lax.broadcasted_iota(jnp.int32, (B, NP), 1)
    col0 = iota == 0
    padc = iota >= N
    iota_f = iota.astype(jnp.float32)
    tlanes = lax.broadcasted_iota(jnp.int32, (B, 256), 1)
    dr = dr_ref[...]
    hu = hu_ref[...]
    hv = hv_ref[...]
    notcol0 = jnp.where(col0, 0.0, 1.0)

    def step(t, carry):
        oh_prev, visited, remaining, prevdep, done, lacc, acts_acc = carry
        g = g_ref[t]
        done_b = done > 0.5
        prevdep_b = prevdep > 0.5

        tm = jnp.sum(t2_ref[...] * oh_prev[:, :, None], axis=1)
        logits = (tm + remaining * hu + hv) / SQRT_D

        nv_c = (1.0 - visited) * notcol0
        all_served = jnp.max(nv_c, axis=1, keepdims=True) < 0.5
        feas = jnp.where(dr <= remaining, 1.0, 0.0) * nv_c
        has_feas = jnp.max(feas, axis=1, keepdims=True) > 0.5

        atdep_b = prevdep_b & (~done_b)
        notdep_b = (~prevdep_b) & (~done_b)
        infeasible = (visited > 0.5) | (dr > remaining)
        mask_depot = atdep_b & has_feas
        infeasible = infeasible | (mask_depot & col0)
        force = (all_served & notdep_b) | (notdep_b & (~has_feas)) | done_b

        ml = jnp.where(padc, -1e30, jnp.where(infeasible, -1e9, logits))
        mx = jnp.max(ml, axis=-1, keepdims=True)
        sh = ml - mx
        logp = sh - jnp.log(jnp.sum(jnp.exp(sh), axis=-1, keepdims=True))

        vals = g + ml
        vm = jnp.max(vals, axis=-1, keepdims=True)
        sel_f = jnp.min(jnp.where(vals == vm, iota_f, float(NP)), axis=-1,
                        keepdims=True)
        sel_f = jnp.where(force, 0.0, sel_f)
        onehot = iota_f == sel_f
        sel_logp = jnp.sum(jnp.where(onehot, logp, 0.0), axis=-1,
                           keepdims=True)
        sel_logp = jnp.where(force, 0.0, sel_logp)

        isdep = sel_f < 0.5
        take = jnp.sum(jnp.where(onehot, dr, 0.0), axis=-1, keepdims=True)
        remaining = jnp.where(isdep, 1.0, remaining - take)
        visited = jnp.maximum(visited, jnp.where(onehot & (~isdep), 1.0, 0.0))
        oh_new = jnp.where(onehot, 1.0, 0.0)
        prevdep = jnp.where(isdep, 1.0, 0.0)
        done = jnp.where(done_b | (all_served & isdep), 1.0, 0.0)
        lacc = lacc + sel_logp

        acts_acc = jnp.where(tlanes == t, sel_f.astype(jnp.int32), acts_acc)
        return (oh_new, visited, remaining, prevdep, done, lacc, acts_acc)

    init = (jnp.where(col0, 1.0, 0.0),
            (iota >= N).astype(jnp.float32),
            jnp.ones((B, 1), jnp.float32),
            jnp.ones((B, 1), jnp.float32),
            jnp.zeros((B, 1), jnp.float32),
            jnp.zeros((B, 1), jnp.float32),
            jnp.zeros((B, 256), jnp.int32))
    final = lax.fori_loop(0, MAX_STEPS, step, init)
    logps_ref[...] = final[5]
    acts_ref[...] = final[6]


def _whole(shape):
    nd = len(shape)
    return pl.BlockSpec(shape, lambda *_: (0,) * nd)


def kernel(coords, demands_raw, capacity_raw, params):
    cap = capacity_raw.reshape(B, 1)
    demand_ratio = demands_raw / cap

    x3 = jnp.concatenate([coords, demand_ratio[..., None]], axis=-1)
    x3p = jnp.zeros((B, NP, 8), jnp.float32).at[:, :N, :3].set(x3)
    wemb = jnp.pad(params['W_embed'], ((0, 0), (0, 5))).T
    bemb = params['b_embed'].reshape(1, D)
    wqkv_t = jnp.transpose(params['Wqkv'], (0, 2, 1))
    wo_t = jnp.transpose(params['Wo'], (0, 2, 1))
    w1_t = jnp.transpose(params['W1'], (0, 2, 1))
    w2_t = jnp.transpose(params['W2'], (0, 2, 1))

    h_pad = pl.pallas_call(
        _enc_kernel,
        grid=(B // BB,),
        in_specs=[
            pl.BlockSpec((BB, NP, 8), lambda i: (i, 0, 0)),
            _whole((8, D)), _whole((1, D)),
            _whole((L, D, 3 * D)), _whole((L, 3 * D)),
            _whole((L, D, D)), _whole((L, D)),
            _whole((L, D, FF)), _whole((L, FF)),
            _whole((L, FF, D)), _whole((L, D)),
            _whole((L, D)), _whole((L, D)),
            _whole((L, D)), _whole((L, D)),
        ],
        out_specs=pl.BlockSpec((BB, NP, D), lambda i: (i, 0, 0)),
        out_shape=jax.ShapeDtypeStruct((B, NP, D), jnp.float32),
    )(x3p, wemb, bemb, wqkv_t, params['bqkv'], wo_t, params['bo'],
      w1_t, params['b1'], w2_t, params['b2'],
      params['ln1_g'], params['ln1_b'], params['ln2_g'], params['ln2_b'])

    t2, hu, hv = pl.pallas_call(
        _pre_kernel,
        grid=(B // PB,),
        in_specs=[
            pl.BlockSpec((PB, NP, D), lambda i: (i, 0, 0)),
            _whole((D, D)), _whole((D, D)), _whole((D, D)),
            _whole((D, D)), _whole((D, D)),
            _whole((1, D)), _whole((1, D)), _whole((1, D)),
        ],
        out_specs=[
            pl.BlockSpec((PB, NP, NP), lambda i: (i, 0, 0)),
            pl.BlockSpec((PB, NP), lambda i: (i, 0)),
            pl.BlockSpec((PB, NP), lambda i: (i, 0)),
        ],
        out_shape=[
            jax.ShapeDtypeStruct((B, NP, NP), jnp.float32),
            jax.ShapeDtypeStruct((B, NP), jnp.float32),
            jax.ShapeDtypeStruct((B, NP), jnp.float32),
        ],
        scratch_shapes=[
            pltpu.VMEM((D, D), jnp.float32),
            pltpu.VMEM((D, D), jnp.float32),
            pltpu.VMEM((8, D), jnp.float32),
        ],
    )(h_pad, params['Wctx'][:, 0:D].T, params['Wctx'][:, D:2 * D].T,
      params['Wctx'][:, 2 * D:3 * D].T, params['Wq'].T, params['Wk'],
      params['bcap'].reshape(1, D), params['bctx'].reshape(1, D),
      params['Wcap'].T)

    keys = jax.random.split(jax.random.key(42), MAX_STEPS)
    G = jax.vmap(lambda k: jax.random.gumbel(k, (B, N), jnp.float32))(keys)
    G_pad = jnp.zeros((MAX_STEPS, B, NP), jnp.float32).at[:, :, :N].set(G)
    dr_pad = jnp.pad(demand_ratio, ((0, 0), (0, NP - N)), constant_values=2.0)

    acts, logps = pl.pallas_call(
        _dec_kernel,
        in_specs=[
            _whole((B, NP, NP)),
            _whole((B, NP)), _whole((B, NP)), _whole((B, NP)),
            _whole((MAX_STEPS, B, NP)),
        ],
        out_specs=[
            _whole((B, 256)),
            _whole((B, 1)),
        ],
        out_shape=[
            jax.ShapeDtypeStruct((B, 256), jnp.int32),
            jax.ShapeDtypeStruct((B, 1), jnp.float32),
        ],
    )(t2, hu, hv, dr_pad, G_pad)

    actions = acts[:, :MAX_STEPS]
    path = jnp.concatenate([jnp.zeros((B, 1), actions.dtype), actions], axis=1)
    return path, logps.reshape(B)


# speedup vs baseline: 1.0631x; 1.0000x over previous
import math
import jax
import jax.numpy as jnp
from jax import lax
from jax.experimental import pallas as pl
from jax.experimental.pallas import tpu as pltpu

B, N, D, H, L = 256, 101, 128, 8, 3
FF = 2048
NP = 128
DH = D // H
MAX_STEPS = 2 * (N - 1) + 1
SQRT_D = math.sqrt(D)
BB = 32
PB = 32


def _ln(x, g, b):
    mu = jnp.mean(x, -1, keepdims=True)
    xc = x - mu
    var = jnp.mean(xc * xc, -1, keepdims=True)
    return xc / jnp.sqrt(var + 1e-5) * g + b


def _enc_kernel(x_ref, wemb_ref, bemb_ref, wqkv_ref, bqkv_ref, wo_ref, bo_ref,
                w1_ref, b1_ref, w2_ref, b2_ref, ln1g_ref, ln1b_ref,
                ln2g_ref, ln2b_ref, h_out_ref):
    M = BB * NP
    x = x_ref[...].reshape(M, 8)
    h = jnp.dot(x, wemb_ref[...]) + bemb_ref[...]
    jmask = lax.broadcasted_iota(jnp.int32, (1, 1, NP), 2) >= N
    for l in range(L):
        qkv = jnp.dot(h, wqkv_ref[l]) + bqkv_ref[l:l + 1, :]
        q3 = qkv[:, 0:D].reshape(BB, NP, D)
        k3 = qkv[:, D:2 * D].reshape(BB, NP, D)
        v3 = qkv[:, 2 * D:3 * D].reshape(BB, NP, D)
        o_parts = []
        for hd in range(H):
            sl = slice(hd * DH, (hd + 1) * DH)
            qh = q3[:, :, sl]
            kh = k3[:, :, sl]
            vh = v3[:, :, sl]
            s = lax.dot_general(qh, kh, (((2,), (2,)), ((0,), (0,))),
                               ) / 4.0
            s = jnp.where(jmask, -1e30, s)
            m = jnp.max(s, axis=-1, keepdims=True)
            e = jnp.exp(s - m)
            att = e / jnp.sum(e, axis=-1, keepdims=True)
            o_parts.append(lax.dot_general(att, vh, (((2,), (1,)), ((0,), (0,))),
                                          ))
        o = jnp.concatenate(o_parts, axis=-1).reshape(M, D)
        o = jnp.dot(o, wo_ref[l]) + bo_ref[l:l + 1, :]
        h = _ln(h + o, ln1g_ref[l:l + 1, :], ln1b_ref[l:l + 1, :])
        ff = b2_ref[l:l + 1, :].astype(jnp.float32) + jnp.zeros((M, D), jnp.float32)
        for c in range(2):
            cs = slice(c * (FF // 2), (c + 1) * (FF // 2))
            ff1c = jnp.maximum(jnp.dot(h, w1_ref[l][:, cs])
                               + b1_ref[l:l + 1, cs], 0.0)
            ff = ff + jnp.dot(ff1c, w2_ref[l][cs, :])
        h = _ln(h + ff, ln2g_ref[l:l + 1, :], ln2b_ref[l:l + 1, :])
    h3 = h.reshape(BB, NP, D)
    rmask = lax.broadcasted_iota(jnp.int32, (1, NP, 1), 1) < N
    h_out_ref[...] = jnp.where(rmask, h3, 0.0)


def _pre_kernel(h_ref, wc1t_ref, wc2t_ref, wc3t_ref, wqt_ref, wk_ref,
                bcap_ref, bctx_ref, wcap_ref,
                t2_ref, hu_ref, hv_ref,
                m1t_s, m2t_s, vrow_s):
    i = pl.program_id(0)

    @pl.when(i == 0)
    def _init():
        p = jnp.dot(wqt_ref[...], wk_ref[...])
        m1t_s[...] = jnp.dot(wc1t_ref[...], p)
        m2t_s[...] = jnp.dot(wc2t_ref[...], p)
        y = (jnp.dot(bcap_ref[...], wc3t_ref[...])
             + bctx_ref[...])
        c0 = jnp.dot(y, p)
        tw = jnp.dot(wcap_ref[...], wc3t_ref[...])
        m3w = jnp.dot(tw, p)
        vrow_s[0:1, :] = c0
        vrow_s[1:2, :] = m3w

    h = h_ref[...]
    gctx = jnp.sum(h, axis=1) / float(N)
    z = jnp.dot(gctx, m2t_s[...]) + vrow_s[0:1, :]
    u2 = jnp.dot(h.reshape(PB * NP, D), m1t_s[...],
                ).reshape(PB, NP, D)
    t2_ref[...] = lax.dot_general(u2, h, (((2,), (2,)), ((0,), (0,))),
                                 )
    hu_ref[...] = jnp.sum(h * vrow_s[1:2, :][None], axis=-1)
    hv_ref[...] = jnp.sum(h * z[:, None, :], axis=-1)


def _dec_kernel(t2_ref, hu_ref, hv_ref, dr_ref, g_ref,
                acts_ref, logps_ref):
    iota = lax.broadcasted_iota(jnp.int32, (B, NP), 1)
    col0 = iota == 0
    padc = iota >= N
    iota_f = iota.astype(jnp.float32)
    tlanes = lax.broadcasted_iota(jnp.int32, (B, 256), 1)
    dr = dr_ref[...]
    hu = hu_ref[...]
    hv = hv_ref[...]
    notcol0 = jnp.where(col0, 0.0, 1.0)

    def step(t, carry):
        oh_prev, visited, remaining, prevdep, done, lacc, acts_acc = carry
        g = g_ref[t]
        done_b = done > 0.5
        prevdep_b = prevdep > 0.5

        tm = jnp.sum(t2_ref[...] * oh_prev[:, :, None], axis=1)
        logits = (tm + remaining * hu + hv) / SQRT_D

        nv_c = (1.0 - visited) * notcol0
        all_served = jnp.max(nv_c, axis=1, keepdims=True) < 0.5
        feas = jnp.where(dr <= remaining, 1.0, 0.0) * nv_c
        has_feas = jnp.max(feas, axis=1, keepdims=True) > 0.5

        atdep_b = prevdep_b & (~done_b)
        notdep_b = (~prevdep_b) & (~done_b)
        infeasible = (visited > 0.5) | (dr > remaining)
        mask_depot = atdep_b & has_feas
        infeasible = infeasible | (mask_depot & col0)
        force = (all_served & notdep_b) | (notdep_b & (~has_feas)) | done_b

        ml = jnp.where(padc, -1e30, jnp.where(infeasible, -1e9, logits))
        mx = jnp.max(ml, axis=-1, keepdims=True)
        sh = ml - mx
        logp = sh - jnp.log(jnp.sum(jnp.exp(sh), axis=-1, keepdims=True))

        vals = g + ml
        vm = jnp.max(vals, axis=-1, keepdims=True)
        sel_f = jnp.min(jnp.where(vals == vm, iota_f, float(NP)), axis=-1,
                        keepdims=True)
        sel_f = jnp.where(force, 0.0, sel_f)
        onehot = iota_f == sel_f
        sel_logp = jnp.sum(jnp.where(onehot, logp, 0.0), axis=-1,
                           keepdims=True)
        sel_logp = jnp.where(force, 0.0, sel_logp)

        isdep = sel_f < 0.5
        take = jnp.sum(jnp.where(onehot, dr, 0.0), axis=-1, keepdims=True)
        remaining = jnp.where(isdep, 1.0, remaining - take)
        visited = jnp.maximum(visited, jnp.where(onehot & (~isdep), 1.0, 0.0))
        oh_new = jnp.where(onehot, 1.0, 0.0)
        prevdep = jnp.where(isdep, 1.0, 0.0)
        done = jnp.where(done_b | (all_served & isdep), 1.0, 0.0)
        lacc = lacc + sel_logp

        acts_acc = jnp.where(tlanes == t, sel_f.astype(jnp.int32), acts_acc)
        return (oh_new, visited, remaining, prevdep, done, lacc, acts_acc)

    init = (jnp.where(col0, 1.0, 0.0),
            (iota >= N).astype(jnp.float32),
            jnp.ones((B, 1), jnp.float32),
            jnp.ones((B, 1), jnp.float32),
            jnp.zeros((B, 1), jnp.float32),
            jnp.zeros((B, 1), jnp.float32),
            jnp.zeros((B, 256), jnp.int32))
    final = lax.fori_loop(0, MAX_STEPS, step, init)
    logps_ref[...] = final[5]
    acts_ref[...] = final[6]


def _whole(shape):
    nd = len(shape)
    return pl.BlockSpec(shape, lambda *_: (0,) * nd)


def kernel(coords, demands_raw, capacity_raw, params):
    cap = capacity_raw.reshape(B, 1)
    demand_ratio = demands_raw / cap

    x3 = jnp.concatenate([coords, demand_ratio[..., None]], axis=-1)
    x3p = jnp.zeros((B, NP, 8), jnp.float32).at[:, :N, :3].set(x3)
    wemb = jnp.pad(params['W_embed'], ((0, 0), (0, 5))).T
    bemb = params['b_embed'].reshape(1, D)
    wqkv_t = jnp.transpose(params['Wqkv'], (0, 2, 1))
    wo_t = jnp.transpose(params['Wo'], (0, 2, 1))
    w1_t = jnp.transpose(params['W1'], (0, 2, 1))
    w2_t = jnp.transpose(params['W2'], (0, 2, 1))

    h_pad = pl.pallas_call(
        _enc_kernel,
        grid=(B // BB,),
        in_specs=[
            pl.BlockSpec((BB, NP, 8), lambda i: (i, 0, 0)),
            _whole((8, D)), _whole((1, D)),
            _whole((L, D, 3 * D)), _whole((L, 3 * D)),
            _whole((L, D, D)), _whole((L, D)),
            _whole((L, D, FF)), _whole((L, FF)),
            _whole((L, FF, D)), _whole((L, D)),
            _whole((L, D)), _whole((L, D)),
            _whole((L, D)), _whole((L, D)),
        ],
        out_specs=pl.BlockSpec((BB, NP, D), lambda i: (i, 0, 0)),
        out_shape=jax.ShapeDtypeStruct((B, NP, D), jnp.float32),
    )(x3p, wemb, bemb, wqkv_t, params['bqkv'], wo_t, params['bo'],
      w1_t, params['b1'], w2_t, params['b2'],
      params['ln1_g'], params['ln1_b'], params['ln2_g'], params['ln2_b'])

    t2, hu, hv = pl.pallas_call(
        _pre_kernel,
        grid=(B // PB,),
        in_specs=[
            pl.BlockSpec((PB, NP, D), lambda i: (i, 0, 0)),
            _whole((D, D)), _whole((D, D)), _whole((D, D)),
            _whole((D, D)), _whole((D, D)),
            _whole((1, D)), _whole((1, D)), _whole((1, D)),
        ],
        out_specs=[
            pl.BlockSpec((PB, NP, NP), lambda i: (i, 0, 0)),
            pl.BlockSpec((PB, NP), lambda i: (i, 0)),
            pl.BlockSpec((PB, NP), lambda i: (i, 0)),
        ],
        out_shape=[
            jax.ShapeDtypeStruct((B, NP, NP), jnp.float32),
            jax.ShapeDtypeStruct((B, NP), jnp.float32),
            jax.ShapeDtypeStruct((B, NP), jnp.float32),
        ],
        scratch_shapes=[
            pltpu.VMEM((D, D), jnp.float32),
            pltpu.VMEM((D, D), jnp.float32),
            pltpu.VMEM((8, D), jnp.float32),
        ],
    )(h_pad, params['Wctx'][:, 0:D].T, params['Wctx'][:, D:2 * D].T,
      params['Wctx'][:, 2 * D:3 * D].T, params['Wq'].T, params['Wk'],
      params['bcap'].reshape(1, D), params['bctx'].reshape(1, D),
      params['Wcap'].T)

    keys = jax.random.split(jax.random.key(42), MAX_STEPS)
    G = jax.vmap(lambda k: jax.random.gumbel(k, (B, N), jnp.float32))(keys)
    G_pad = jnp.zeros((MAX_STEPS, B, NP), jnp.float32).at[:, :, :N].set(G)
    dr_pad = jnp.pad(demand_ratio, ((0, 0), (0, NP - N)), constant_values=2.0)

    acts, logps = pl.pallas_call(
        _dec_kernel,
        in_specs=[
            _whole((B, NP, NP)),
            _whole((B, NP)), _whole((B, NP)), _whole((B, NP)),
            _whole((MAX_STEPS, B, NP)),
        ],
        out_specs=[
            _whole((B, 256)),
            _whole((B, 1)),
        ],
        out_shape=[
            jax.ShapeDtypeStruct((B, 256), jnp.int32),
            jax.ShapeDtypeStruct((B, 1), jnp.float32),
        ],
    )(t2, hu, hv, dr_pad, G_pad)

    actions = acts[:, :MAX_STEPS]
    path = jnp.concatenate([jnp.zeros((B, 1), actions.dtype), actions], axis=1)
    return path, logps.reshape(B)
